# baseline (speedup 1.0000x reference)
import functools

import jax
import jax.numpy as jnp
from jax import lax
from jax.experimental import pallas as pl
from jax.experimental.pallas import tpu as pltpu
from jax.experimental.pallas import tpu_sc as plsc

_B = 1024
_D = 64
_N_NEG = 5
_TOTAL = _B * (2 + _N_NEG)

_NC = 2
_NS = 16
_NW = _NC * _NS
_RPW = _TOTAL // _NW
_BAT = 112
_NB = _RPW // _BAT

_TN = 512
_NT = (_TOTAL - _B) // _TN
_POS_T = _B // _TN


def _sc_gather(table, idx):
    mesh = plsc.VectorSubcoreMesh(core_axis_name="c", subcore_axis_name="s")

    @functools.partial(
        pl.kernel,
        out_type=jax.ShapeDtypeStruct((_TOTAL, _D), jnp.float32),
        mesh=mesh,
        scratch_types=[
            pltpu.VMEM((_RPW,), jnp.int32),
            pltpu.VMEM_SHARED((_NS * _RPW, _D), jnp.float32),
            pltpu.SemaphoreType.DMA,
        ],
    )
    def gather_k(table_hbm, idx_hbm, out_hbm, idx_v, stage, gsem):
        cid = lax.axis_index("c")
        sid = lax.axis_index("s")
        base = (cid * _NS + sid) * _RPW
        sbase = sid * _RPW
        pltpu.sync_copy(idx_hbm.at[pl.ds(base, _RPW)], idx_v)
        gcopies = []
        for g in range(_RPW // 16):
            vec = idx_v[pl.ds(g * 16, 16)]
            for l in range(16):
                gcopies.append(
                    pltpu.async_copy(
                        table_hbm.at[pl.ds(vec[l], 1)],
                        stage.at[pl.ds(sbase + g * 16 + l, 1)],
                        gsem,
                    )
                )
        for c in gcopies:
            c.wait()
        pltpu.sync_copy(
            stage.at[pl.ds(sbase, _RPW)], out_hbm.at[pl.ds(base, _RPW)]
        )

    return gather_k(table, idx)


def _tc_body(lhs_ref, rhs_ref, out_ref):
    i = pl.program_id(0)
    logits = lax.dot_general(
        lhs_ref[...], rhs_ref[...],
        (((1,), (1,)), ((), ())),
        preferred_element_type=jnp.float32,
    )
    is_pos = i < _POS_T
    sign = jnp.where(is_pos, -1.0, 1.0).astype(jnp.float32)
    x = logits * sign
    tile_sum = jnp.sum(jnp.maximum(x, 0.0) + jnp.log1p(jnp.exp(-jnp.abs(x))))
    w = jnp.where(
        is_pos, 0.5 / (_B * _B), 0.5 / (_B * _B * _N_NEG)
    ).astype(jnp.float32)

    @pl.when(i == 0)
    def _():
        out_ref[...] = jnp.zeros_like(out_ref)

    out_ref[...] += jnp.full((1, 1), tile_sum * w, jnp.float32)


def _tc_loss(rows):
    return pl.pallas_call(
        _tc_body,
        grid=(_NT,),
        in_specs=[
            pl.BlockSpec((_B, _D), lambda i: (0, 0)),
            pl.BlockSpec((_TN, _D), lambda i: (i + _POS_T, 0)),
        ],
        out_specs=pl.BlockSpec((1, 1), lambda i: (0, 0)),
        out_shape=jax.ShapeDtypeStruct((1, 1), jnp.float32),
    )(rows, rows)


def kernel(src, pos, neg, table):
    idx = jnp.concatenate([src, pos, neg.reshape(-1)])
    rows = _sc_gather(table, idx)
    return _tc_loss(rows)[0, 0]


# speedup vs baseline: 1.0201x; 1.0201x over previous
import functools

import jax
import jax.numpy as jnp
from jax import lax
from jax.experimental import pallas as pl
from jax.experimental.pallas import tpu as pltpu
from jax.experimental.pallas import tpu_sc as plsc

_B = 1024
_D = 64
_N_NEG = 5
_TOTAL = _B * (2 + _N_NEG)

_NC = 2
_NS = 16
_NW = _NC * _NS
_RPW = _TOTAL // _NW
_HALF = _RPW // 2

_TN = 512
_NT = (_TOTAL - _B) // _TN
_POS_T = _B // _TN


def _sc_gather(table, idx):
    mesh = plsc.VectorSubcoreMesh(core_axis_name="c", subcore_axis_name="s")

    @functools.partial(
        pl.kernel,
        out_type=jax.ShapeDtypeStruct((_TOTAL, _D), jnp.float32),
        mesh=mesh,
        scratch_types=[
            pltpu.VMEM((_RPW,), jnp.int32),
            pltpu.VMEM((_HALF, _D), jnp.float32),
            pltpu.VMEM_SHARED((_NS * _HALF, _D), jnp.float32),
            pltpu.SemaphoreType.DMA,
            pltpu.SemaphoreType.DMA,
        ],
    )
    def gather_k(table_hbm, idx_hbm, out_hbm, idx_v, buf, stage, gsem, dsem):
        cid = lax.axis_index("c")
        sid = lax.axis_index("s")
        base = (cid * _NS + sid) * _RPW
        sbase = sid * _HALF
        pltpu.sync_copy(idx_hbm.at[pl.ds(base, _RPW)], idx_v)
        vecs = [idx_v[pl.ds(g * 16, 16)] for g in range(_RPW // 16)]
        copies = []
        for j in range(_HALF):
            sv = vecs[j // 16]
            dv = vecs[_HALF // 16 + j // 16]
            l = j % 16
            copies.append(
                pltpu.async_copy(
                    table_hbm.at[pl.ds(sv[l], 1)],
                    buf.at[pl.ds(j, 1)],
                    gsem,
                )
            )
            copies.append(
                pltpu.async_copy(
                    table_hbm.at[pl.ds(dv[l], 1)],
                    stage.at[pl.ds(sbase + j, 1)],
                    dsem,
                )
            )
        for c in copies:
            c.wait()
        w1 = pltpu.async_copy(buf, out_hbm.at[pl.ds(base, _HALF)], gsem)
        w2 = pltpu.async_copy(
            stage.at[pl.ds(sbase, _HALF)],
            out_hbm.at[pl.ds(base + _HALF, _HALF)],
            dsem,
        )
        w1.wait()
        w2.wait()

    return gather_k(table, idx)


def _tc_body(lhs_ref, rhs_ref, out_ref):
    i = pl.program_id(0)
    logits = lax.dot_general(
        lhs_ref[...], rhs_ref[...],
        (((1,), (1,)), ((), ())),
        preferred_element_type=jnp.float32,
    )
    is_pos = i < _POS_T
    sign = jnp.where(is_pos, -1.0, 1.0).astype(jnp.float32)
    x = logits * sign
    tile_sum = jnp.sum(jnp.maximum(x, 0.0) + jnp.log1p(jnp.exp(-jnp.abs(x))))
    w = jnp.where(
        is_pos, 0.5 / (_B * _B), 0.5 / (_B * _B * _N_NEG)
    ).astype(jnp.float32)

    @pl.when(i == 0)
    def _():
        out_ref[...] = jnp.zeros_like(out_ref)

    out_ref[...] += jnp.full((1, 1), tile_sum * w, jnp.float32)


def _tc_loss(rows):
    return pl.pallas_call(
        _tc_body,
        grid=(_NT,),
        in_specs=[
            pl.BlockSpec((_B, _D), lambda i: (0, 0)),
            pl.BlockSpec((_TN, _D), lambda i: (i + _POS_T, 0)),
        ],
        out_specs=pl.BlockSpec((1, 1), lambda i: (0, 0)),
        out_shape=jax.ShapeDtypeStruct((1, 1), jnp.float32),
    )(rows, rows)


def kernel(src, pos, neg, table):
    idx = jnp.concatenate([src, pos, neg.reshape(-1)])
    rows = _sc_gather(table, idx)
    return _tc_loss(rows)[0, 0]
